# Initial kernel scaffold; baseline (speedup 1.0000x reference)
import jax
import jax.numpy as jnp
from jax.experimental import pallas as pl


def kernel(x, edge_index, W0, b0, W1, b1):
    raise NotImplementedError("write your pallas kernel here")


# speedup vs baseline: 2.7489x; 2.7489x over previous
import functools

import jax
import jax.numpy as jnp
from jax import lax
from jax.experimental import pallas as pl
from jax.experimental.pallas import tpu as pltpu
from jax.experimental.pallas import tpu_sc as plsc

_N = 10000
_D = 128
_E = 320000
_K = 10
_ALPHA = 0.1

_NC = 2
_NS = 16
_CH = 128
_JPW = 157
_EPAD = _NS * _JPW * _CH
_OWN = 5120
_R = _NC * _OWN
_R2 = 6144
_JUNK = 5632
_ZC = _R2 // (_NS * _CH)
_WC = 5
_WR = _OWN // (_NS * _WC)

_mesh = plsc.VectorSubcoreMesh(core_axis_name="c", subcore_axis_name="s")


def _agg_body(h_hbm, src_hbm, dst_hbm, out_hbm,
              src_v, dst_v, rows_v, zeros_v, wb_v, acc_sh, sem):
    c = lax.axis_index("c")
    s = lax.axis_index("s")
    zbase = s * (_ZC * _CH)
    wbase = s * (_WC * _WR)
    rbase = c * _OWN

    def zfill(i, _):
        for l in range(_D // 16):
            zeros_v[i, pl.ds(l * 16, 16)] = jnp.zeros((16,), jnp.float32)
        return 0
    lax.fori_loop(0, _CH, zfill, 0)

    def zchunk(i, _):
        pltpu.sync_copy(zeros_v, acc_sh.at[pl.ds(zbase + i * _CH, _CH)])
        return 0
    lax.fori_loop(0, _ZC, zchunk, 0)

    pltpu.sync_copy(src_hbm.at[s], src_v)
    pltpu.sync_copy(dst_hbm.at[s], dst_v)

    def remap(j, _):
        for l in range(_CH // 16):
            t = dst_v[j, pl.ds(l * 16, 16)] - rbase
            bad = (t < 0) | (t >= _OWN)
            dst_v[j, pl.ds(l * 16, 16)] = jnp.where(bad, _JUNK, t)
        return 0
    lax.fori_loop(0, _JPW, remap, 0)
    plsc.subcore_barrier()

    def step(j, _):
        pltpu.async_copy(h_hbm.at[src_v.at[j]], rows_v, sem).wait()
        pltpu.sync_copy(rows_v, acc_sh.at[dst_v.at[j]], add=True)
        return 0
    lax.fori_loop(0, _JPW, step, 0)
    plsc.subcore_barrier()

    def wchunk(i, _):
        pltpu.sync_copy(acc_sh.at[pl.ds(wbase + i * _WR, _WR)], wb_v)
        pltpu.sync_copy(wb_v, out_hbm.at[c, pl.ds(wbase + i * _WR, _WR)])
        return 0
    lax.fori_loop(0, _WC, wchunk, 0)


_agg_call = functools.partial(
    pl.kernel,
    out_type=jax.ShapeDtypeStruct((_NC, _OWN, _D), jnp.float32),
    mesh=_mesh,
    scratch_types=[
        pltpu.VMEM((_JPW, _CH), jnp.int32),
        pltpu.VMEM((_JPW, _CH), jnp.int32),
        pltpu.VMEM((_CH, _D), jnp.float32),
        pltpu.VMEM((_CH, _D), jnp.float32),
        pltpu.VMEM((_WR, _D), jnp.float32),
        pltpu.VMEM_SHARED((_R2, _D), jnp.float32),
        pltpu.SemaphoreType.DMA,
    ],
)(_agg_body)


_BLK = 512


def _combine_body(a, av, bv, o):
    o[...] = a[...] * av[...] + bv[...]


def _combine_call(a, av, bv):
    return pl.pallas_call(
        _combine_body,
        grid=(_R // _BLK,),
        in_specs=[pl.BlockSpec((_BLK, _D), lambda i: (i, 0))] * 3,
        out_specs=pl.BlockSpec((_BLK, _D), lambda i: (i, 0)),
        out_shape=jax.ShapeDtypeStruct((_R, _D), jnp.float32),
    )(a, av, bv)


def _mlp_body(hv, rv, w0, b0, w1, b1, o):
    h = hv[...] * rv[...]
    z = jnp.maximum(
        jnp.dot(h, w0[...], preferred_element_type=jnp.float32) + b0[...], 0.0)
    o[...] = jnp.dot(z, w1[...], preferred_element_type=jnp.float32) + b1[...]


def _mlp_call(hv, rv, w0, b0, w1, b1):
    H = w0.shape[1]
    P = w1.shape[1]
    return pl.pallas_call(
        _mlp_body,
        grid=(_R // _BLK,),
        in_specs=[
            pl.BlockSpec((_BLK, _D), lambda i: (i, 0)),
            pl.BlockSpec((_BLK, _D), lambda i: (i, 0)),
            pl.BlockSpec((_D, H), lambda i: (0, 0)),
            pl.BlockSpec((1, H), lambda i: (0, 0)),
            pl.BlockSpec((H, P), lambda i: (0, 0)),
            pl.BlockSpec((1, P), lambda i: (0, 0)),
        ],
        out_specs=pl.BlockSpec((_BLK, P), lambda i: (i, 0)),
        out_shape=jax.ShapeDtypeStruct((_R, P), jnp.float32),
    )(hv, rv, w0, b0, w1, b1)


def kernel(x, edge_index, W0, b0, W1, b1):
    src = edge_index[0].astype(jnp.int32)
    dst = edge_index[1].astype(jnp.int32)
    pad = jnp.full((_EPAD - _E,), _N, jnp.int32)
    src3 = jnp.concatenate([src, pad]).reshape(_NS, _JPW, _CH)
    dst3 = jnp.concatenate([dst, pad]).reshape(_NS, _JPW, _CH)

    ones_h = jnp.zeros((_R, _D), jnp.float32).at[:_N].set(1.0)
    dparts = _agg_call(ones_h, src3, dst3)
    deg_in = dparts.reshape(_R, _D)[:_N, 0]
    ones_h2 = ones_h + 0.0 * dparts.reshape(_R, _D)
    dparts = _agg_call(ones_h2, dst3, src3)
    deg_out = dparts.reshape(_R, _D)[:_N, 0]
    in_norm = lax.rsqrt(jnp.maximum(deg_in, 1.0))
    out_norm = lax.rsqrt(jnp.maximum(deg_out, 1.0))

    zpad = jnp.zeros((_R - _N,), jnp.float32)
    opad = jnp.ones((_R - _N,), jnp.float32)
    onp = jnp.concatenate([out_norm, opad])
    inp_ = jnp.concatenate([in_norm, zpad])
    xpad = jnp.pad(x, ((0, _R - _N), (0, 0)))

    scale = (1.0 - _ALPHA)
    av = jnp.broadcast_to((scale * onp * inp_)[:, None], (_R, _D))
    bv = (_ALPHA * onp)[:, None] * xpad
    recip = jnp.concatenate([jnp.sqrt(jnp.maximum(deg_out, 1.0)), opad])
    rv = jnp.broadcast_to(recip[:, None], (_R, _D))

    def step(_, h):
        parts = _agg_call(h, src3, dst3)
        return _combine_call(parts.reshape(_R, _D), av, bv)

    h = lax.fori_loop(0, _K, step, onp[:, None] * xpad)

    H = W0.shape[1]
    C = W1.shape[1]
    P = 128
    W1p = jnp.pad(W1, ((0, 0), (0, P - C)))
    b1p = jnp.pad(b1, (0, P - C)).reshape(1, P)
    b0r = b0.reshape(1, H)
    logits = _mlp_call(h, rv, W0, b0r, W1p, b1p)
    return logits[:_N, :C]


# speedup vs baseline: 4.0550x; 1.4751x over previous
import functools

import jax
import jax.numpy as jnp
from jax import lax
from jax.experimental import pallas as pl
from jax.experimental.pallas import tpu as pltpu
from jax.experimental.pallas import tpu_sc as plsc

_N = 10000
_D = 128
_E = 320000
_K = 10
_ALPHA = 0.1

_NC = 2
_NS = 16
_DH = _D // _NC
_CH = 128
_JPW = 157
_EPAD = _NS * _JPW * _CH
_R = 10240
_ZC = _R // (_NS * _CH)

_mesh = plsc.VectorSubcoreMesh(core_axis_name="c", subcore_axis_name="s")


def _agg_body(h_hbm, src_hbm, dst_hbm, out_hbm,
              src_v, dst_v, rows_v, zeros_v, wb_v, acc_sh, sem):
    c = lax.axis_index("c")
    s = lax.axis_index("s")
    base = s * (_ZC * _CH)

    def zfill(i, _):
        for l in range(_DH // 16):
            zeros_v[i, pl.ds(l * 16, 16)] = jnp.zeros((16,), jnp.float32)
        return 0
    lax.fori_loop(0, _CH, zfill, 0)

    def zchunk(i, _):
        pltpu.sync_copy(zeros_v, acc_sh.at[pl.ds(base + i * _CH, _CH)])
        return 0
    lax.fori_loop(0, _ZC, zchunk, 0)

    pltpu.sync_copy(src_hbm.at[s], src_v)
    pltpu.sync_copy(dst_hbm.at[s], dst_v)
    plsc.subcore_barrier()

    def step(j, _):
        pltpu.async_copy(h_hbm.at[c].at[src_v.at[j]], rows_v, sem).wait()
        pltpu.sync_copy(rows_v, acc_sh.at[dst_v.at[j]], add=True)
        return 0
    lax.fori_loop(0, _JPW, step, 0)
    plsc.subcore_barrier()

    def wchunk(i, _):
        pltpu.sync_copy(acc_sh.at[pl.ds(base + i * _CH, _CH)], wb_v)
        pltpu.sync_copy(wb_v, out_hbm.at[c, pl.ds(base + i * _CH, _CH)])
        return 0
    lax.fori_loop(0, _ZC, wchunk, 0)


_agg_call = functools.partial(
    pl.kernel,
    out_type=jax.ShapeDtypeStruct((_NC, _R, _DH), jnp.float32),
    mesh=_mesh,
    compiler_params=pltpu.CompilerParams(use_tc_tiling_on_sc=False),
    scratch_types=[
        pltpu.VMEM((_JPW, _CH), jnp.int32),
        pltpu.VMEM((_JPW, _CH), jnp.int32),
        pltpu.VMEM((_CH, _DH), jnp.float32),
        pltpu.VMEM((_CH, _DH), jnp.float32),
        pltpu.VMEM((_CH, _DH), jnp.float32),
        pltpu.VMEM_SHARED((_R, _DH), jnp.float32),
        pltpu.SemaphoreType.DMA,
    ],
)(_agg_body)


_BLK = 512


def _combine_body(a, av, bv, o):
    o[...] = a[...] * av[...] + bv[...]


def _combine_call(a, av, bv):
    spec = pl.BlockSpec((_NC, _BLK, _DH), lambda i: (0, i, 0))
    return pl.pallas_call(
        _combine_body,
        grid=(_R // _BLK,),
        in_specs=[spec] * 3,
        out_specs=spec,
        out_shape=jax.ShapeDtypeStruct((_NC, _R, _DH), jnp.float32),
    )(a, av, bv)


def _mlp_body(hv, rv, w0, b0, w1, b1, o):
    g = hv[...] * rv[...]
    h = jnp.concatenate([g[0], g[1]], axis=-1)
    z = jnp.maximum(
        jnp.dot(h, w0[...], preferred_element_type=jnp.float32) + b0[...], 0.0)
    o[...] = jnp.dot(z, w1[...], preferred_element_type=jnp.float32) + b1[...]


def _mlp_call(hv, rv, w0, b0, w1, b1):
    H = w0.shape[1]
    P = w1.shape[1]
    return pl.pallas_call(
        _mlp_body,
        grid=(_R // _BLK,),
        in_specs=[
            pl.BlockSpec((_NC, _BLK, _DH), lambda i: (0, i, 0)),
            pl.BlockSpec((_NC, _BLK, _DH), lambda i: (0, i, 0)),
            pl.BlockSpec((_D, H), lambda i: (0, 0)),
            pl.BlockSpec((1, H), lambda i: (0, 0)),
            pl.BlockSpec((H, P), lambda i: (0, 0)),
            pl.BlockSpec((1, P), lambda i: (0, 0)),
        ],
        out_specs=pl.BlockSpec((_BLK, P), lambda i: (i, 0)),
        out_shape=jax.ShapeDtypeStruct((_R, P), jnp.float32),
    )(hv, rv, w0, b0, w1, b1)


def _to_split(m):
    return m.reshape(_R, _NC, _DH).transpose(1, 0, 2)


def kernel(x, edge_index, W0, b0, W1, b1):
    src = edge_index[0].astype(jnp.int32)
    dst = edge_index[1].astype(jnp.int32)
    pad = jnp.full((_EPAD - _E,), _N, jnp.int32)
    src3 = jnp.concatenate([src, pad]).reshape(_NS, _JPW, _CH)
    dst3 = jnp.concatenate([dst, pad]).reshape(_NS, _JPW, _CH)

    ones_h = jnp.zeros((_NC, _R, _DH), jnp.float32).at[:, :_N].set(1.0)
    dparts = _agg_call(ones_h, src3, dst3)
    deg_in = dparts[0, :_N, 0]
    ones_h2 = ones_h + 0.0 * dparts
    dparts = _agg_call(ones_h2, dst3, src3)
    deg_out = dparts[0, :_N, 0]
    in_norm = lax.rsqrt(jnp.maximum(deg_in, 1.0))
    out_norm = lax.rsqrt(jnp.maximum(deg_out, 1.0))

    zpad = jnp.zeros((_R - _N,), jnp.float32)
    opad = jnp.ones((_R - _N,), jnp.float32)
    onp = jnp.concatenate([out_norm, opad])
    inp_ = jnp.concatenate([in_norm, zpad])
    xpad = jnp.pad(x, ((0, _R - _N), (0, 0)))

    scale = (1.0 - _ALPHA)
    av = _to_split(jnp.broadcast_to((scale * onp * inp_)[:, None], (_R, _D)))
    bv = _to_split((_ALPHA * onp)[:, None] * xpad)
    recip = jnp.concatenate([jnp.sqrt(jnp.maximum(deg_out, 1.0)), opad])
    rv = _to_split(jnp.broadcast_to(recip[:, None], (_R, _D)))

    def step(_, h):
        return _combine_call(_agg_call(h, src3, dst3), av, bv)

    h = lax.fori_loop(0, _K, step, _to_split(onp[:, None] * xpad))

    H = W0.shape[1]
    C = W1.shape[1]
    P = 128
    W1p = jnp.pad(W1, ((0, 0), (0, P - C)))
    b1p = jnp.pad(b1, (0, P - C)).reshape(1, P)
    b0r = b0.reshape(1, H)
    logits = _mlp_call(h, rv, W0, b0r, W1p, b1p)
    return logits[:_N, :C]
